# baseline (speedup 1.0000x reference)
import functools

import jax
import jax.numpy as jnp
from jax import lax
from jax.experimental import pallas as pl
from jax.experimental.pallas import tpu as pltpu
from jax.experimental.pallas import tpu_sc as plsc

N = 10000
NPAD = 10240
KNBR = 16
NG = 16
NCLS = 10
NW = 32
PW = NPAD // NW
INF = float(3.0e38)


def _knn_body(px_h, py_h, pz_h, g0_h, g1_h, idx_out, inp1_out,
              pxv, pyv, pzv, g0v, g1v, idxloc, inp1loc, dall, icmp, mtab):
    wid = lax.axis_index("s") * 2 + lax.axis_index("c")
    base = wid * PW
    pltpu.sync_copy(px_h, pxv.at[pl.ds(0, NPAD)])
    pltpu.sync_copy(py_h, pyv.at[pl.ds(0, NPAD)])
    pltpu.sync_copy(pz_h, pzv.at[pl.ds(0, NPAD)])
    pltpu.sync_copy(g0_h.at[pl.ds(base, PW)], g0v.at[pl.ds(0, PW)])
    pltpu.sync_copy(g1_h.at[pl.ds(base, PW)], g1v.at[pl.ds(0, PW)])

    lane = jnp.arange(16, dtype=jnp.int32)
    lanestr = lane * (PW * 6)

    @pl.loop(0, PW)
    def _minit(p):
        mtab[p, :] = jnp.full((16,), INF, jnp.float32)

    def _run(pstart):
        g0 = g0v[pl.ds(pstart, 16)][0]
        g1 = g1v[pl.ds(pstart, 16)][0]
        pe = jnp.maximum(jnp.minimum(jnp.int32(PW), g1 - base), pstart + 1)
        s0 = (g0 // 16) * 16
        nblk = (g1 - s0 + 127) // 128

        def blk(cb, _):
            sb = s0 + cb * 128
            cands = []
            for u in range(8):
                st = sb + u * 16
                li = st + lane
                ok = (li >= g0) & (li < g1)
                cands.append((
                    jnp.where(ok, pxv[pl.ds(st, 16)], 1e19),
                    pyv[pl.ds(st, 16)],
                    pzv[pl.ds(st, 16)],
                ))

            def inner(pp, _):
                xi = pxv[pl.ds(base + pp, 16)][0]
                yi = pyv[pl.ds(base + pp, 16)][0]
                zi = pzv[pl.ds(base + pp, 16)][0]
                m = mtab[pp, :]
                for cx, cy, cz in cands:
                    dx = cx - xi
                    dy = cy - yi
                    dz = cz - zi
                    m = jnp.minimum(m, (dx * dx + dy * dy) + dz * dz)
                mtab[pp, :] = m
                return 0

            lax.fori_loop(pstart, pe, inner, 0)
            return 0

        lax.fori_loop(0, nblk, blk, 0)
        return pe

    lax.while_loop(lambda pp: pp < PW, _run, jnp.int32(0))

    @pl.loop(0, PW)
    def _point(p):
        i = base + p
        xi = pxv[pl.ds(i, 16)][0]
        yi = pyv[pl.ds(i, 16)][0]
        zi = pzv[pl.ds(i, 16)][0]
        g0 = g0v[pl.ds(p, 16)][0]
        g1 = g1v[pl.ds(p, 16)][0]
        s0 = (g0 // 16) * 16
        nt = (g1 - s0 + 15) // 16
        ntb = (nt + 3) // 4

        tau = jnp.max(mtab[p, :])

        def pass_b(tb, cur):
            for u in range(4):
                t = tb * 4 + u
                st = s0 + t * 16
                li = st + lane
                cx = jnp.where((li >= g0) & (li < g1),
                               pxv[pl.ds(st, 16)], 1e19)
                dx = cx - xi
                dy = pyv[pl.ds(st, 16)] - yi
                dz = pzv[pl.ds(st, 16)] - zi
                d = (dx * dx + dy * dy) + dz * dz
                m = d <= tau
                plsc.store_compressed(dall.at[pl.ds(cur, 16)], d, mask=m)
                plsc.store_compressed(icmp.at[pl.ds(cur, 16)], li, mask=m)
                cur = cur + plsc.all_reduce_population_count(m)[0]
            return cur

        ncand = lax.fori_loop(0, ntb, pass_b, jnp.int32(0))

        best_d0 = jnp.full((16,), INF, jnp.float32)
        best_i0 = jnp.full((16,), i, jnp.int32)

        def pass_c(c, carry):
            best_d, best_i = carry
            d = dall[pl.ds(c * 16, 16)]
            ii = icmp[pl.ds(c * 16, 16)]
            d = jnp.where(c * 16 + lane < ncand, d, INF)
            dk, iv = plsc.sort_key_val(d, ii)
            rd = lax.rev(best_d, (0,))
            ri = lax.rev(best_i, (0,))
            take = dk <= rd
            lo_d = jnp.minimum(dk, rd)
            lo_i = jnp.where(take, iv, ri)
            nd, ni = plsc.sort_key_val(lo_d, lo_i)
            return (nd, ni)

        best_d, best_i = lax.fori_loop(0, (ncand + 15) // 16, pass_c,
                                       (best_d0, best_i0))

        gx = plsc.load_gather(pxv, [best_i])
        gy = plsc.load_gather(pyv, [best_i])
        gz = plsc.load_gather(pzv, [best_i])
        plsc.store_scatter(idxloc, [lane * PW + p], best_i)
        fb = p * 6
        plsc.store_scatter(inp1loc, [fb + lanestr + 0], gx)
        plsc.store_scatter(inp1loc, [fb + lanestr + 1], gy)
        plsc.store_scatter(inp1loc, [fb + lanestr + 2], gz)
        plsc.store_scatter(inp1loc, [fb + lanestr + 3], gx - xi)
        plsc.store_scatter(inp1loc, [fb + lanestr + 4], gy - yi)
        plsc.store_scatter(inp1loc, [fb + lanestr + 5], gz - zi)

    for k in range(KNBR):
        pltpu.sync_copy(idxloc.at[pl.ds(k * PW, PW)],
                        idx_out.at[pl.ds(k * NPAD + base, PW)])
        pltpu.sync_copy(inp1loc.at[pl.ds(k * PW * 6, PW * 6)],
                        inp1_out.at[pl.ds((k * NPAD + base) * 6, PW * 6)])


def _knn_call(px, py, pz, g0pp, g1pp):
    mesh = plsc.VectorSubcoreMesh(core_axis_name="c", subcore_axis_name="s")
    return pl.kernel(
        _knn_body,
        mesh=mesh,
        compiler_params=pltpu.CompilerParams(
            needs_layout_passes=False, skip_device_barrier=True),
        out_type=[
            jax.ShapeDtypeStruct((KNBR * NPAD,), jnp.int32),
            jax.ShapeDtypeStruct((KNBR * NPAD * 6,), jnp.float32),
        ],
        scratch_types=[
            pltpu.VMEM((NPAD + 16,), jnp.float32),
            pltpu.VMEM((NPAD + 16,), jnp.float32),
            pltpu.VMEM((NPAD + 16,), jnp.float32),
            pltpu.VMEM((PW + 16,), jnp.int32),
            pltpu.VMEM((PW + 16,), jnp.int32),
            pltpu.VMEM((KNBR * PW,), jnp.int32),
            pltpu.VMEM((PW * 96,), jnp.float32),
            pltpu.VMEM((NPAD + 80,), jnp.float32),
            pltpu.VMEM((NPAD + 80,), jnp.int32),
            pltpu.VMEM((PW, 16), jnp.float32),
        ],
    )(px, py, pz, g0pp, g1pp)


EDGES = NPAD * KNBR
EW = EDGES // NW
CH = 128
NCH = EW // CH
GRP = 2
HW = 128
NGRPS = NCH // GRP


def _gather_body(h_h, idx3_h, hj_out, idx2d, idxsh, bufa, bufb, hjc,
                 sema, semb):
    wid = lax.axis_index("s") * 2 + lax.axis_index("c")
    base = wid * EW
    lane = jnp.arange(16, dtype=jnp.int32)
    pltpu.sync_copy(idx3_h.at[wid], idx2d)

    @pl.loop(0, NCH)
    def _shift(c):
        for q in range(CH // 16):
            idxsh[c, pl.ds(q * 16, 16)] = (
                idx2d[c, pl.ds(q * 16, 16)] >> 2)

    bufs = (bufa, bufb)
    sems = (sema, semb)

    def fire(g):
        buf, sem = bufs[g % 2], sems[g % 2]
        return [pltpu.async_copy(h_h.at[idxsh.at[g * GRP + j]],
                                 buf.at[pl.ds(j * CH, CH)], sem)
                for j in range(GRP)]

    descs = {0: fire(0)}
    for g in range(NGRPS):
        if g + 1 < NGRPS:
            descs[g + 1] = fire(g + 1)
        for d in descs.pop(g):
            d.wait()
        buf = bufs[g % 2]

        @pl.loop(0, GRP * CH // 16)
        def _extract(q):
            c = g * GRP + q // (CH // 16)
            jv = idx2d[c, pl.ds((q % (CH // 16)) * 16, 16)]
            rows = q * 16 + lane
            cols = (jv & 3) * 32
            r32 = rows * 32
            for f in range(32):
                vals = plsc.load_gather(buf, [rows, cols + f])
                plsc.store_scatter(hjc, [r32 + f], vals)

        pltpu.sync_copy(
            hjc, hj_out.at[pl.ds((base + g * GRP * CH) * 32, GRP * CH * 32)])


def _gather_call(h, idx3):
    mesh = plsc.VectorSubcoreMesh(core_axis_name="c", subcore_axis_name="s")
    return pl.kernel(
        _gather_body,
        mesh=mesh,
        compiler_params=pltpu.CompilerParams(
            needs_layout_passes=False, skip_device_barrier=True),
        out_type=jax.ShapeDtypeStruct((EDGES * 32,), jnp.float32),
        scratch_types=[
            pltpu.VMEM((NCH, CH), jnp.int32),
            pltpu.VMEM((NCH, CH), jnp.int32),
            pltpu.VMEM((GRP * CH, HW), jnp.float32),
            pltpu.VMEM((GRP * CH, HW), jnp.float32),
            pltpu.VMEM((GRP * CH * 32,), jnp.float32),
            pltpu.SemaphoreType.DMA,
            pltpu.SemaphoreType.DMA,
        ],
    )(h, idx3)


BP = 256
BP4 = BP // 4
NBLK = NPAD // BP


def _layer1_body(inp_ref, w1_ref, b1_ref, w2_ref, b2_ref, h_ref):
    x = inp_ref[...].reshape(KNBR * BP4, 24)
    a = jnp.dot(x, w1_ref[...], preferred_element_type=jnp.float32)
    a = jnp.maximum(a + b1_ref[...], 0.0)
    m = jnp.dot(a, w2_ref[...], preferred_element_type=jnp.float32)
    m = m + b2_ref[...]
    m = jnp.max(m.reshape(KNBR, BP4, 128), axis=0)
    h_ref[...] = jnp.maximum(m, 0.0)


def _layer1_call(inp1, w1bd, b1t, w2bd, b2t):
    return pl.pallas_call(
        _layer1_body,
        grid=(NBLK,),
        in_specs=[
            pl.BlockSpec((KNBR, BP4, 24), lambda i: (0, i, 0)),
            pl.BlockSpec((24, 128), lambda i: (0, 0)),
            pl.BlockSpec((1, 128), lambda i: (0, 0)),
            pl.BlockSpec((128, 128), lambda i: (0, 0)),
            pl.BlockSpec((1, 128), lambda i: (0, 0)),
        ],
        out_specs=pl.BlockSpec((BP4, 128), lambda i: (i, 0)),
        out_shape=jax.ShapeDtypeStruct((NPAD // 4, 128), jnp.float32),
    )(inp1, w1bd, b1t, w2bd, b2t)


def _layer2_body(hj_ref, inp_ref, bat_ref, w2h_ref, w2r_ref, b2a_ref,
                 w2b_ref, b2b_ref, wc_ref, bc_ref, out_ref, pooled_ref):
    step = pl.program_id(0)

    @pl.when(step == 0)
    def _():
        pooled_ref[...] = jnp.full((NG, 128), -jnp.inf, jnp.float32)

    hj = hj_ref[...].reshape(KNBR * BP4, 128)
    x24 = inp_ref[...].reshape(KNBR * BP4, 24)
    a = (jnp.dot(hj, w2h_ref[...], preferred_element_type=jnp.float32)
         + jnp.dot(x24, w2r_ref[...], preferred_element_type=jnp.float32))
    a = jnp.maximum(a + b2a_ref[...], 0.0)
    m = jnp.dot(a, w2b_ref[...], preferred_element_type=jnp.float32)
    m = m + b2b_ref[...]
    m = jnp.max(m.reshape(KNBR, BP4, 128), axis=0)
    h2 = jnp.maximum(m, 0.0)

    bat = bat_ref[0]
    gids = lax.broadcasted_iota(jnp.int32, (BP4, NG, 128), 1)
    b3 = jnp.broadcast_to(bat.reshape(BP4, 1, 128), (BP4, NG, 128))
    h3 = jnp.broadcast_to(h2.reshape(BP4, 1, 128), (BP4, NG, 128))
    contrib = jnp.max(jnp.where(gids == b3, h3, -jnp.inf), axis=0)
    pooled_ref[...] = jnp.maximum(pooled_ref[...], contrib)

    @pl.when(step == NBLK - 1)
    def _():
        pooled = jnp.max(pooled_ref[...].reshape(NG, 4, 32), axis=1)
        out_ref[...] = (jnp.dot(pooled, wc_ref[...],
                                preferred_element_type=jnp.float32)
                        + bc_ref[...])


def _layer2_call(hj, inp1, bat3, w2hbd, w2rbd, b2at, w2bbd, b2bt, wct, bc):
    return pl.pallas_call(
        _layer2_body,
        grid=(NBLK,),
        in_specs=[
            pl.BlockSpec((KNBR, BP4, 128), lambda i: (0, i, 0)),
            pl.BlockSpec((KNBR, BP4, 24), lambda i: (0, i, 0)),
            pl.BlockSpec((1, BP4, 128), lambda i: (i, 0, 0)),
            pl.BlockSpec((128, 128), lambda i: (0, 0)),
            pl.BlockSpec((24, 128), lambda i: (0, 0)),
            pl.BlockSpec((1, 128), lambda i: (0, 0)),
            pl.BlockSpec((128, 128), lambda i: (0, 0)),
            pl.BlockSpec((1, 128), lambda i: (0, 0)),
            pl.BlockSpec((32, NCLS), lambda i: (0, 0)),
            pl.BlockSpec((1, NCLS), lambda i: (0, 0)),
        ],
        out_specs=pl.BlockSpec((NG, NCLS), lambda i: (0, 0)),
        out_shape=jax.ShapeDtypeStruct((NG, NCLS), jnp.float32),
        scratch_shapes=[pltpu.VMEM((NG, 128), jnp.float32)],
    )(hj, inp1, bat3, w2hbd, w2rbd, b2at, w2bbd, b2bt, wct, bc)


def kernel(pos, batch, W1a, b1a, W1b, b1b, W2a, b2a, W2b, b2b, Wc, bc):
    batch = batch.astype(jnp.int32)
    pad = NPAD - N
    gidx = jnp.arange(NG, dtype=jnp.int32)
    starts = jnp.searchsorted(batch, gidx, side="left").astype(jnp.int32)
    ends = jnp.searchsorted(batch, gidx, side="right").astype(jnp.int32)
    g0pp = jnp.pad(starts[batch], (0, pad))
    g1pp = jnp.pad(ends[batch], (0, pad))
    px = jnp.pad(pos[:, 0], (0, pad))
    py = jnp.pad(pos[:, 1], (0, pad))
    pz = jnp.pad(pos[:, 2], (0, pad))

    idx, inp1_flat = _knn_call(px, py, pz, g0pp, g1pp)
    inp1 = inp1_flat.reshape(KNBR, NPAD // 4, 24)

    eye4 = jnp.eye(4, dtype=jnp.float32)
    kron = lambda w: jnp.kron(eye4, w)
    tile4 = lambda b: jnp.tile(b.reshape(1, -1), (1, 4))
    h = _layer1_call(inp1, kron(W1a.T), tile4(b1a), kron(W1b.T), tile4(b1b))

    idx3 = idx.reshape(NW, NCH, CH)
    hj = _gather_call(h, idx3).reshape(KNBR, NPAD // 4, 128)

    batch_pad = jnp.pad(batch, (0, pad), constant_values=-1)
    bat128 = jnp.repeat(batch_pad, 32).reshape(NBLK, BP4, 128)
    w2r6 = jnp.concatenate(
        [jnp.zeros((3, 32), jnp.float32), W2a[:, 32:].T], axis=0)
    out = _layer2_call(hj, inp1, bat128, kron(W2a[:, :32].T), kron(w2r6),
                       tile4(b2a), kron(W2b.T), tile4(b2b),
                       Wc.T, bc.reshape(1, NCLS))
    return out


# speedup vs baseline: 1.3234x; 1.3234x over previous
import functools

import jax
import jax.numpy as jnp
from jax import lax
from jax.experimental import pallas as pl
from jax.experimental.pallas import tpu as pltpu
from jax.experimental.pallas import tpu_sc as plsc

N = 10000
NPAD = 10240
KNBR = 16
NG = 16
NCLS = 10
NW = 32
PW = NPAD // NW
INF = float(3.0e38)
CCAP = 5120


def _knn_body(px_h, py_h, pz_h, g0_h, g1_h, idx_out, inp1_out,
              pxv, pyv, pzv, g0v, g1v, idxloc, inp1loc, dall, icmp, mtab):
    wid = lax.axis_index("s") * 2 + lax.axis_index("c")
    base = wid * PW
    pltpu.sync_copy(px_h, pxv.at[pl.ds(0, NPAD)])
    pltpu.sync_copy(py_h, pyv.at[pl.ds(0, NPAD)])
    pltpu.sync_copy(pz_h, pzv.at[pl.ds(0, NPAD)])
    pltpu.sync_copy(g0_h.at[pl.ds(base, PW)], g0v.at[pl.ds(0, PW)])
    pltpu.sync_copy(g1_h.at[pl.ds(base, PW)], g1v.at[pl.ds(0, PW)])

    lane = jnp.arange(16, dtype=jnp.int32)
    lanestr = lane * (PW * 8)

    @pl.loop(0, PW)
    def _minit(p):
        mtab[p, :] = jnp.full((16,), INF, jnp.float32)

    def _run(pstart):
        g0 = g0v[pl.ds(pstart, 16)][0]
        g1 = g1v[pl.ds(pstart, 16)][0]
        pe = jnp.maximum(jnp.minimum(jnp.int32(PW), g1 - base), pstart + 1)
        s0 = (g0 // 16) * 16
        nblk = (g1 - s0 + 127) // 128

        def blk(cb, _):
            sb = s0 + cb * 128
            cands = []
            for u in range(8):
                st = sb + u * 16
                li = st + lane
                ok = (li >= g0) & (li < g1)
                cands.append((
                    jnp.where(ok, pxv[pl.ds(st, 16)], 1e19),
                    pyv[pl.ds(st, 16)],
                    pzv[pl.ds(st, 16)],
                ))

            def inner(pp, _):
                xi = pxv[pl.ds(base + pp, 16)][0]
                yi = pyv[pl.ds(base + pp, 16)][0]
                zi = pzv[pl.ds(base + pp, 16)][0]
                m = mtab[pp, :]
                for cx, cy, cz in cands:
                    dx = cx - xi
                    dy = cy - yi
                    dz = cz - zi
                    m = jnp.minimum(m, (dx * dx + dy * dy) + dz * dz)
                mtab[pp, :] = m
                return 0

            lax.fori_loop(pstart, pe, inner, 0)
            return 0

        lax.fori_loop(0, nblk, blk, 0)
        return pe

    lax.while_loop(lambda pp: pp < PW, _run, jnp.int32(0))

    @pl.loop(0, PW)
    def _point(p):
        i = base + p
        xi = pxv[pl.ds(i, 16)][0]
        yi = pyv[pl.ds(i, 16)][0]
        zi = pzv[pl.ds(i, 16)][0]
        g0 = g0v[pl.ds(p, 16)][0]
        g1 = g1v[pl.ds(p, 16)][0]
        s0 = (g0 // 16) * 16
        nt = (g1 - s0 + 15) // 16
        ntb = (nt + 3) // 4

        tau = jnp.max(mtab[p, :])

        def pass_b(tb, cur):
            for u in range(4):
                t = tb * 4 + u
                st = s0 + t * 16
                li = st + lane
                cx = jnp.where((li >= g0) & (li < g1),
                               pxv[pl.ds(st, 16)], 1e19)
                dx = cx - xi
                dy = pyv[pl.ds(st, 16)] - yi
                dz = pzv[pl.ds(st, 16)] - zi
                d = (dx * dx + dy * dy) + dz * dz
                m = d <= tau
                cw = jnp.minimum(cur, jnp.int32(CCAP))
                plsc.store_compressed(dall.at[pl.ds(cw, 16)], d, mask=m)
                plsc.store_compressed(icmp.at[pl.ds(cw, 16)], li, mask=m)
                cur = cur + plsc.all_reduce_population_count(m)[0]
            return cur

        ncand = lax.fori_loop(0, ntb, pass_b, jnp.int32(0))

        best_d0 = jnp.full((16,), INF, jnp.float32)
        best_i0 = jnp.full((16,), i, jnp.int32)

        def _merge(d, ii, best_d, best_i):
            dk, iv = plsc.sort_key_val(d, ii)
            rd = lax.rev(best_d, (0,))
            ri = lax.rev(best_i, (0,))
            take = dk <= rd
            lo_d = jnp.minimum(dk, rd)
            lo_i = jnp.where(take, iv, ri)
            nd, ni = plsc.sort_key_val(lo_d, lo_i)
            return (nd, ni)

        def _fast(bd, bi):
            def pass_c(c, carry):
                best_d, best_i = carry
                d = dall[pl.ds(c * 16, 16)]
                ii = icmp[pl.ds(c * 16, 16)]
                d = jnp.where(c * 16 + lane < ncand, d, INF)
                return _merge(d, ii, best_d, best_i)

            return lax.fori_loop(0, (ncand + 15) // 16, pass_c, (bd, bi))

        def _slow(bd, bi):
            def ch(t, carry):
                best_d, best_i = carry
                st = s0 + t * 16
                li = st + lane
                cx = jnp.where((li >= g0) & (li < g1),
                               pxv[pl.ds(st, 16)], 1e19)
                dx = cx - xi
                dy = pyv[pl.ds(st, 16)] - yi
                dz = pzv[pl.ds(st, 16)] - zi
                d = (dx * dx + dy * dy) + dz * dz
                return _merge(d, li, best_d, best_i)

            return lax.fori_loop(0, nt, ch, (bd, bi))

        best_d, best_i = lax.cond(ncand > CCAP, _slow, _fast,
                                  best_d0, best_i0)

        gx = plsc.load_gather(pxv, [best_i])
        gy = plsc.load_gather(pyv, [best_i])
        gz = plsc.load_gather(pzv, [best_i])
        plsc.store_scatter(idxloc, [lane * PW + p], best_i)
        fb = p * 8
        zz = jnp.zeros((16,), jnp.float32)
        plsc.store_scatter(inp1loc, [fb + lanestr + 0], gx)
        plsc.store_scatter(inp1loc, [fb + lanestr + 1], gy)
        plsc.store_scatter(inp1loc, [fb + lanestr + 2], gz)
        plsc.store_scatter(inp1loc, [fb + lanestr + 3], gx - xi)
        plsc.store_scatter(inp1loc, [fb + lanestr + 4], gy - yi)
        plsc.store_scatter(inp1loc, [fb + lanestr + 5], gz - zi)
        plsc.store_scatter(inp1loc, [fb + lanestr + 6], zz)
        plsc.store_scatter(inp1loc, [fb + lanestr + 7], zz)

    for k in range(KNBR):
        pltpu.sync_copy(idxloc.at[pl.ds(k * PW, PW)],
                        idx_out.at[pl.ds(k * NPAD + base, PW)])
        pltpu.sync_copy(inp1loc.at[pl.ds(k * PW * 8, PW * 8)],
                        inp1_out.at[pl.ds((k * NPAD + base) * 8, PW * 8)])


def _knn_call(px, py, pz, g0pp, g1pp):
    mesh = plsc.VectorSubcoreMesh(core_axis_name="c", subcore_axis_name="s")
    return pl.kernel(
        _knn_body,
        mesh=mesh,
        compiler_params=pltpu.CompilerParams(
            needs_layout_passes=False, skip_device_barrier=True),
        out_type=[
            jax.ShapeDtypeStruct((KNBR * NPAD,), jnp.int32),
            jax.ShapeDtypeStruct((KNBR * NPAD * 8,), jnp.float32),
        ],
        scratch_types=[
            pltpu.VMEM((NPAD + 16,), jnp.float32),
            pltpu.VMEM((NPAD + 16,), jnp.float32),
            pltpu.VMEM((NPAD + 16,), jnp.float32),
            pltpu.VMEM((PW + 16,), jnp.int32),
            pltpu.VMEM((PW + 16,), jnp.int32),
            pltpu.VMEM((KNBR * PW,), jnp.int32),
            pltpu.VMEM((KNBR * PW * 8,), jnp.float32),
            pltpu.VMEM((CCAP + 96,), jnp.float32),
            pltpu.VMEM((CCAP + 96,), jnp.int32),
            pltpu.VMEM((PW, 16), jnp.float32),
        ],
    )(px, py, pz, g0pp, g1pp)


EDGES = NPAD * KNBR
EW = EDGES // NW
CH = 128
NCH = EW // CH
GRP = 2
HW = 128
NGRPS = NCH // GRP


def _gather_body(h_h, idx3_h, hj_out, idx2d, bufa, bufb, hjc, sema, semb):
    wid = lax.axis_index("s") * 2 + lax.axis_index("c")
    base = wid * EW
    pltpu.sync_copy(idx3_h.at[wid], idx2d)
    bufs = (bufa, bufb)
    sems = (sema, semb)

    def fire(g):
        buf, sem = bufs[g % 2], sems[g % 2]
        return [pltpu.async_copy(h_h.at[idx2d.at[g * GRP + j]],
                                 buf.at[pl.ds(j * CH, CH)], sem)
                for j in range(GRP)]

    descs = {0: fire(0)}
    for g in range(NGRPS):
        if g + 1 < NGRPS:
            descs[g + 1] = fire(g + 1)
        for d in descs.pop(g):
            d.wait()
        buf = bufs[g % 2]

        @pl.loop(0, GRP * CH, unroll=8)
        def _row(r):
            hjc[pl.ds(r * 32, 16)] = buf[r, pl.ds(0, 16)]
            hjc[pl.ds(r * 32 + 16, 16)] = buf[r, pl.ds(16, 16)]

        pltpu.sync_copy(
            hjc, hj_out.at[pl.ds((base + g * GRP * CH) * 32, GRP * CH * 32)])


def _gather_call(h, idx3):
    mesh = plsc.VectorSubcoreMesh(core_axis_name="c", subcore_axis_name="s")
    return pl.kernel(
        _gather_body,
        mesh=mesh,
        compiler_params=pltpu.CompilerParams(skip_device_barrier=True),
        out_type=jax.ShapeDtypeStruct((EDGES * 32,), jnp.float32),
        scratch_types=[
            pltpu.VMEM((NCH, CH), jnp.int32),
            pltpu.VMEM((GRP * CH, HW), jnp.float32),
            pltpu.VMEM((GRP * CH, HW), jnp.float32),
            pltpu.VMEM((GRP * CH * 32,), jnp.float32),
            pltpu.SemaphoreType.DMA,
            pltpu.SemaphoreType.DMA,
        ],
    )(h, idx3)


BP = 256
BP4 = BP // 4
BP16 = BP // 16
NBLK = NPAD // BP


def _layer1_body(inp_ref, w1_ref, b1_ref, w2_ref, b2_ref, h_ref):
    x = inp_ref[...].reshape(KNBR * BP16, 128)
    a = jnp.dot(x, w1_ref[...], preferred_element_type=jnp.float32)
    a = jnp.maximum(a + b1_ref[...], 0.0)
    a = a.reshape(KNBR * BP16, 4, 128).reshape(KNBR * BP4, 128)
    m = jnp.dot(a, w2_ref[...], preferred_element_type=jnp.float32)
    m = m + b2_ref[...]
    m = jnp.max(m.reshape(KNBR, BP4, 128), axis=0)
    h4 = jnp.maximum(m, 0.0)
    h = jnp.stack([h4[:, 32 * c:32 * (c + 1)] for c in range(4)],
                  axis=1).reshape(BP, 32)
    h_ref[...] = jnp.concatenate(
        [h, jnp.zeros((BP, HW - 32), jnp.float32)], axis=1)


def _layer1_call(inp1, w1bd, b1t, w2bd, b2t):
    return pl.pallas_call(
        _layer1_body,
        grid=(NBLK,),
        in_specs=[
            pl.BlockSpec((KNBR, BP16, 128), lambda i: (0, i, 0)),
            pl.BlockSpec((128, 512), lambda i: (0, 0)),
            pl.BlockSpec((1, 512), lambda i: (0, 0)),
            pl.BlockSpec((128, 128), lambda i: (0, 0)),
            pl.BlockSpec((1, 128), lambda i: (0, 0)),
        ],
        out_specs=pl.BlockSpec((BP, HW), lambda i: (i, 0)),
        out_shape=jax.ShapeDtypeStruct((NPAD, HW), jnp.float32),
    )(inp1, w1bd, b1t, w2bd, b2t)


def _layer2_body(hj_ref, inp_ref, bat_ref, w2h_ref, w2r_ref, b2a_ref,
                 w2b_ref, b2b_ref, wc_ref, bc_ref, out_ref, pooled_ref):
    step = pl.program_id(0)

    @pl.when(step == 0)
    def _():
        pooled_ref[...] = jnp.full((NG, 128), -jnp.inf, jnp.float32)

    hj = hj_ref[...].reshape(KNBR * BP4, 128)
    x8 = inp_ref[...].reshape(KNBR * BP16, 128)
    r = jnp.dot(x8, w2r_ref[...], preferred_element_type=jnp.float32)
    r = r.reshape(KNBR * BP16, 4, 128).reshape(KNBR * BP4, 128)
    a = jnp.dot(hj, w2h_ref[...], preferred_element_type=jnp.float32) + r
    a = jnp.maximum(a + b2a_ref[...], 0.0)
    m = jnp.dot(a, w2b_ref[...], preferred_element_type=jnp.float32)
    m = m + b2b_ref[...]
    m = jnp.max(m.reshape(KNBR, BP4, 128), axis=0)
    h2 = jnp.maximum(m, 0.0)

    bat = bat_ref[0]
    gids = lax.broadcasted_iota(jnp.int32, (BP4, NG, 128), 1)
    b3 = jnp.broadcast_to(bat.reshape(BP4, 1, 128), (BP4, NG, 128))
    h3 = jnp.broadcast_to(h2.reshape(BP4, 1, 128), (BP4, NG, 128))
    contrib = jnp.max(jnp.where(gids == b3, h3, -jnp.inf), axis=0)
    pooled_ref[...] = jnp.maximum(pooled_ref[...], contrib)

    @pl.when(step == NBLK - 1)
    def _():
        pooled = jnp.max(pooled_ref[...].reshape(NG, 4, 32), axis=1)
        out_ref[...] = (jnp.dot(pooled, wc_ref[...],
                                preferred_element_type=jnp.float32)
                        + bc_ref[...])


def _layer2_call(hj, inp1, bat3, w2hbd, w2rbd, b2at, w2bbd, b2bt, wct, bc):
    return pl.pallas_call(
        _layer2_body,
        grid=(NBLK,),
        in_specs=[
            pl.BlockSpec((KNBR, BP4, 128), lambda i: (0, i, 0)),
            pl.BlockSpec((KNBR, BP16, 128), lambda i: (0, i, 0)),
            pl.BlockSpec((1, BP4, 128), lambda i: (i, 0, 0)),
            pl.BlockSpec((128, 128), lambda i: (0, 0)),
            pl.BlockSpec((128, 512), lambda i: (0, 0)),
            pl.BlockSpec((1, 128), lambda i: (0, 0)),
            pl.BlockSpec((128, 128), lambda i: (0, 0)),
            pl.BlockSpec((1, 128), lambda i: (0, 0)),
            pl.BlockSpec((32, NCLS), lambda i: (0, 0)),
            pl.BlockSpec((1, NCLS), lambda i: (0, 0)),
        ],
        out_specs=pl.BlockSpec((NG, NCLS), lambda i: (0, 0)),
        out_shape=jax.ShapeDtypeStruct((NG, NCLS), jnp.float32),
        scratch_shapes=[pltpu.VMEM((NG, 128), jnp.float32)],
    )(hj, inp1, bat3, w2hbd, w2rbd, b2at, w2bbd, b2bt, wct, bc)


def kernel(pos, batch, W1a, b1a, W1b, b1b, W2a, b2a, W2b, b2b, Wc, bc):
    batch = batch.astype(jnp.int32)
    pad = NPAD - N
    gidx = jnp.arange(NG, dtype=jnp.int32)
    starts = jnp.searchsorted(batch, gidx, side="left").astype(jnp.int32)
    ends = jnp.searchsorted(batch, gidx, side="right").astype(jnp.int32)
    g0pp = jnp.pad(starts[batch], (0, pad))
    g1pp = jnp.pad(ends[batch], (0, pad))
    px = jnp.pad(pos[:, 0], (0, pad))
    py = jnp.pad(pos[:, 1], (0, pad))
    pz = jnp.pad(pos[:, 2], (0, pad))

    idx, inp1_flat = _knn_call(px, py, pz, g0pp, g1pp)
    inp1 = inp1_flat.reshape(KNBR, NPAD // 16, 128)

    eye4 = jnp.eye(4, dtype=jnp.float32)
    eye16 = jnp.eye(16, dtype=jnp.float32)
    kron = lambda w: jnp.kron(eye4, w)
    kron16 = lambda w: jnp.kron(eye16, w)
    tile4 = lambda b: jnp.tile(b.reshape(1, -1), (1, 4))
    tile16 = lambda b: jnp.tile(b.reshape(1, -1), (1, 16))
    w1a8 = jnp.concatenate([W1a.T, jnp.zeros((2, 32), jnp.float32)], axis=0)
    h = _layer1_call(inp1, kron16(w1a8), tile16(b1a), kron(W1b.T), tile4(b1b))

    idx3 = idx.reshape(NW, NCH, CH)
    hj = _gather_call(h, idx3).reshape(KNBR, NPAD // 4, 128)

    batch_pad = jnp.pad(batch, (0, pad), constant_values=-1)
    bat128 = jnp.repeat(batch_pad, 32).reshape(NBLK, BP4, 128)
    w2r8 = jnp.concatenate(
        [jnp.zeros((3, 32), jnp.float32), W2a[:, 32:].T,
         jnp.zeros((2, 32), jnp.float32)], axis=0)
    out = _layer2_call(hj, inp1, bat128, kron(W2a[:, :32].T), kron16(w2r8),
                       tile4(b2a), kron(W2b.T), tile4(b2b),
                       Wc.T, bc.reshape(1, NCLS))
    return out
